# baseline (speedup 1.0000x reference)
import functools

import jax
import jax.numpy as jnp
from jax import lax
from jax.experimental import pallas as pl
from jax.experimental.pallas import tpu as pltpu
from jax.experimental.pallas import tpu_sc as plsc

F32 = jnp.float32
C = 21
B = 4
S = 2048
EMB = 512


def _sc_gather(table, ids_flat):
    info = plsc.get_sparse_core_info()
    nw = info.num_cores * info.num_subcores
    n, d = ids_flat.shape[0], table.shape[1]
    b_per_w = n // nw
    ch = 128
    n_ch = b_per_w // ch
    mesh = plsc.VectorSubcoreMesh(core_axis_name="c", subcore_axis_name="s")

    @functools.partial(
        pl.kernel,
        out_type=jax.ShapeDtypeStruct((n, d), F32),
        mesh=mesh,
        scratch_types=[
            pltpu.VMEM((ch,), jnp.int32),
            pltpu.VMEM((ch, d), F32),
            pltpu.SemaphoreType.DMA,
        ],
    )
    def gather_kernel(table_hbm, idx_hbm, out_hbm, idx_v, rows_v, sem):
        wid = lax.axis_index("s") * info.num_cores + lax.axis_index("c")
        for c in range(n_ch):
            base = wid * b_per_w + c * ch
            pltpu.sync_copy(idx_hbm.at[pl.ds(base, ch)], idx_v)
            pltpu.async_copy(table_hbm.at[idx_v], rows_v, sem).wait()
            pltpu.sync_copy(rows_v, out_hbm.at[pl.ds(base, ch)])

    return gather_kernel(table, ids_flat)


def _net_body(x_ref, w1_ref, b1_ref, w2_ref, b2_ref, w3_ref, b3_ref,
              wd_ref, bd_ref, out_ref, xp1, xp2, xp3):
    TL = 256
    NT = S // TL
    for r in range(NT):
        r0 = r * TL
        xp1[pl.ds(r0, TL), :] = x_ref[0, pl.ds(r0, TL), :]
    xp1[pl.ds(S, 1), :] = jnp.zeros((1, EMB), F32)
    xp2[pl.ds(0, 1), :] = jnp.zeros((1, 256), F32)
    xp2[pl.ds(S + 1, 1), :] = jnp.zeros((1, 256), F32)
    xp3[pl.ds(0, 3), :] = jnp.zeros((3, 256), F32)
    xp3[pl.ds(S + 3, 3), :] = jnp.zeros((3, 256), F32)

    b1 = b1_ref[...]
    b2 = b2_ref[...]
    b3 = b3_ref[...]
    bd = bd_ref[...]
    wd = wd_ref[...]

    for r in range(NT):
        r0 = r * TL
        acc = jnp.dot(xp1[pl.ds(r0, TL), :], w1_ref[0],
                      preferred_element_type=F32)
        acc += jnp.dot(xp1[pl.ds(r0 + 1, TL), :], w1_ref[1],
                       preferred_element_type=F32)
        xp2[pl.ds(1 + r0, TL), :] = jnp.maximum(acc + b1, 0.0)
    for r in range(NT):
        r0 = r * TL
        acc = jnp.dot(xp2[pl.ds(r0, TL), :], w2_ref[0],
                      preferred_element_type=F32)
        for w in range(1, 3):
            acc += jnp.dot(xp2[pl.ds(r0 + w, TL), :], w2_ref[w],
                           preferred_element_type=F32)
        xp3[pl.ds(3 + r0, TL), :] = jnp.maximum(acc + b2, 0.0)
    for r in range(NT):
        r0 = r * TL
        acc = jnp.dot(xp3[pl.ds(r0, TL), :], w3_ref[0],
                      preferred_element_type=F32)
        for w in range(1, 4):
            acc += jnp.dot(xp3[pl.ds(r0 + 2 * w, TL), :], w3_ref[w],
                           preferred_element_type=F32)
        h = jnp.maximum(acc + b3, 0.0)
        lg = jnp.dot(h, wd, preferred_element_type=F32)
        out_ref[0, pl.ds(r0, TL), :] = jnp.maximum(lg + bd, 0.0)


def _net(x, w1, b1, w2, b2, w3, b3, wd, bd):
    return pl.pallas_call(
        _net_body,
        grid=(B,),
        in_specs=[
            pl.BlockSpec((1, S, EMB), lambda b: (b, 0, 0)),
            pl.BlockSpec((2, EMB, 256), lambda b: (0, 0, 0)),
            pl.BlockSpec((1, 256), lambda b: (0, 0)),
            pl.BlockSpec((3, 256, 256), lambda b: (0, 0, 0)),
            pl.BlockSpec((1, 256), lambda b: (0, 0)),
            pl.BlockSpec((4, 256, EMB), lambda b: (0, 0, 0)),
            pl.BlockSpec((1, EMB), lambda b: (0, 0)),
            pl.BlockSpec((EMB, C), lambda b: (0, 0)),
            pl.BlockSpec((1, C), lambda b: (0, 0)),
        ],
        out_specs=pl.BlockSpec((1, S, C), lambda b: (b, 0, 0)),
        out_shape=jax.ShapeDtypeStruct((B, S, C), F32),
        scratch_shapes=[
            pltpu.VMEM((S + 1, EMB), F32),
            pltpu.VMEM((S + 2, 256), F32),
            pltpu.VMEM((S + 6, 256), F32),
        ],
        compiler_params=pltpu.CompilerParams(
            dimension_semantics=("arbitrary",)),
    )(x, w1, b1, w2, b2, w3, b3, wd, bd)


def _crf_body(lg_ref, lab_ref, labn_ref, tr_ref, loss_ref, f1_ref, elg_ref):
    BC = B * C
    trm = tr_ref[...]

    CH = 128
    iota_c = lax.broadcasted_iota(jnp.int32, (CH, C), 1)

    def chunk_step(c, carry):
        emis, pairs, msum, tp, fp, fn = carry
        t0 = c * CH
        es = []
        for b in range(B):
            lgc = lg_ref[b, pl.ds(t0, CH), :]
            l0 = lab_ref[b, pl.ds(t0, CH), :]
            l1 = labn_ref[b, pl.ds(t0, CH), :]
            o0 = (l0 == iota_c)
            o1f = jnp.where(l1 == iota_c, 1.0, 0.0)
            o0f = jnp.where(o0, 1.0, 0.0)
            emis += jnp.sum(jnp.where(o0, lgc, 0.0))
            rowv = jnp.dot(o0f, trm, preferred_element_type=F32)
            pairs += jnp.sum(rowv * o1f)
            mx = jnp.max(lgc, axis=1, keepdims=True)
            pred = jnp.min(jnp.where(lgc == mx, iota_c, jnp.int32(10 ** 9)),
                           axis=1, keepdims=True)
            pf = jnp.where(pred == iota_c, 1.0, 0.0)
            tp += jnp.sum(pf * o0f, axis=0, keepdims=True)
            fp += jnp.sum(pf * (1.0 - o0f), axis=0, keepdims=True)
            fn += jnp.sum((1.0 - pf) * o0f, axis=0, keepdims=True)
            es.append(jnp.exp(lgc - mx))
            msum += jnp.sum(mx)
        elg_ref[pl.ds(t0, CH), :] = jnp.concatenate(es, axis=1)
        return emis, pairs, msum, tp, fp, fn

    zrow = jnp.zeros((1, C), F32)
    emis, pairs, msum, tp, fp, fn = lax.fori_loop(
        0, S // CH, chunk_step,
        (jnp.float32(0.0), jnp.float32(0.0), jnp.float32(0.0),
         zrow, zrow, zrow))

    E = jnp.exp(trm)
    E_rows = jnp.concatenate([E] * B, axis=1)
    E44 = jnp.concatenate([E_rows] * B, axis=0)
    segr = lax.broadcasted_iota(jnp.int32, (BC, BC), 0) // C
    segc = lax.broadcasted_iota(jnp.int32, (BC, BC), 1) // C
    E84 = jnp.where(segr == segc, E44, 0.0)
    ones_seg = jnp.where(
        lax.broadcasted_iota(jnp.int32, (BC, B), 0) // C
        == lax.broadcasted_iota(jnp.int32, (BC, B), 1), 1.0, 0.0)
    spread_seg = jnp.where(
        lax.broadcasted_iota(jnp.int32, (B, BC), 0)
        == lax.broadcasted_iota(jnp.int32, (B, BC), 1) // C, 1.0, 0.0)

    K = 16
    n_grp = (S - 1) // K

    def scan_iter(i, carry):
        p, mlog = carry
        t0 = 1 + i * K
        for k in range(K):
            q = jnp.dot(p, E84, preferred_element_type=F32)
            p = q * elg_ref[pl.ds(t0 + k, 1), :]
        s = jnp.dot(p, ones_seg, preferred_element_type=F32)
        p = p / jnp.dot(s, spread_seg, preferred_element_type=F32)
        return p, mlog + jnp.log(s)

    p0 = elg_ref[pl.ds(0, 1), :]
    p, mlog = lax.fori_loop(0, n_grp, scan_iter,
                            (p0, jnp.zeros((1, B), F32)))
    for t in range(1 + n_grp * K, S):
        q = jnp.dot(p, E84, preferred_element_type=F32)
        p = q * elg_ref[pl.ds(t, 1), :]
    tot = jnp.dot(p, ones_seg, preferred_element_type=F32)
    logz_sum = jnp.sum(mlog + jnp.log(tot)) + msum

    loss_ref[0, 0] = (logz_sum - emis - pairs) / B
    p = tp / (tp + fp + 1e-07)
    r = tp / (tp + fn + 1e-07)
    f1 = 2.0 * p * r / (p + r + 1e-07)
    f1_ref[0, 0] = jnp.sum(f1) / C


def _crf(lg, lab3, labn3, trans):
    return pl.pallas_call(
        _crf_body,
        in_specs=[
            pl.BlockSpec((B, S, C), lambda: (0, 0, 0)),
            pl.BlockSpec((B, S, 1), lambda: (0, 0, 0)),
            pl.BlockSpec((B, S, 1), lambda: (0, 0, 0)),
            pl.BlockSpec((C, C), lambda: (0, 0)),
        ],
        out_specs=[
            pl.BlockSpec(memory_space=pltpu.SMEM),
            pl.BlockSpec(memory_space=pltpu.SMEM),
        ],
        out_shape=[
            jax.ShapeDtypeStruct((1, 1), F32),
            jax.ShapeDtypeStruct((1, 1), F32),
        ],
        scratch_shapes=[pltpu.VMEM((S, B * C), F32)],
    )(lg, lab3, labn3, trans)


def kernel(emb_table, conv1_k, conv1_b, conv2_k, conv2_b, idcnn_k, idcnn_b,
           dense_W, dense_b, trans, token_id, label):
    ids = token_id.reshape(B * S).astype(jnp.int32)
    emb = _sc_gather(emb_table, ids)
    x = emb.reshape(B, S, EMB)
    lg = _net(x, conv1_k, conv1_b.reshape(1, -1), conv2_k,
              conv2_b.reshape(1, -1), idcnn_k, idcnn_b.reshape(1, -1),
              dense_W, dense_b.reshape(1, -1))
    lab3 = label.astype(jnp.int32).reshape(B, S, 1)
    labn3 = jnp.concatenate(
        [label.astype(jnp.int32)[:, 1:],
         jnp.full((B, 1), C, jnp.int32)], axis=1).reshape(B, S, 1)
    loss2, f12 = _crf(lg, lab3, labn3, trans)
    return loss2[0, 0], f12[0, 0]


# speedup vs baseline: 2.0229x; 2.0229x over previous
import functools

import jax
import jax.numpy as jnp
from jax import lax
from jax.experimental import pallas as pl
from jax.experimental.pallas import tpu as pltpu
from jax.experimental.pallas import tpu_sc as plsc

F32 = jnp.float32
C = 21
B = 4
S = 2048
EMB = 512


def _sc_gather(table, ids_flat):
    info = plsc.get_sparse_core_info()
    nw = info.num_cores * info.num_subcores
    n, d = ids_flat.shape[0], table.shape[1]
    b_per_w = n // nw
    ch = 128
    n_ch = b_per_w // ch
    mesh = plsc.VectorSubcoreMesh(core_axis_name="c", subcore_axis_name="s")

    @functools.partial(
        pl.kernel,
        out_type=jax.ShapeDtypeStruct((n, d), F32),
        mesh=mesh,
        scratch_types=[
            pltpu.VMEM((ch,), jnp.int32),
            pltpu.VMEM((ch, d), F32),
            pltpu.SemaphoreType.DMA,
        ],
    )
    def gather_kernel(table_hbm, idx_hbm, out_hbm, idx_v, rows_v, sem):
        wid = lax.axis_index("s") * info.num_cores + lax.axis_index("c")
        for c in range(n_ch):
            base = wid * b_per_w + c * ch
            pltpu.sync_copy(idx_hbm.at[pl.ds(base, ch)], idx_v)
            pltpu.async_copy(table_hbm.at[idx_v], rows_v, sem).wait()
            pltpu.sync_copy(rows_v, out_hbm.at[pl.ds(base, ch)])

    return gather_kernel(table, ids_flat)


def _net_body(x_ref, w1_ref, b1_ref, w2_ref, b2_ref, w3_ref, b3_ref,
              wd_ref, bd_ref, out_ref, xp1, xp2, xp3):
    TL = 256
    NT = S // TL
    for r in range(NT):
        r0 = r * TL
        xp1[pl.ds(r0, TL), :] = x_ref[0, pl.ds(r0, TL), :]
    xp1[pl.ds(S, 1), :] = jnp.zeros((1, EMB), F32)
    xp2[pl.ds(0, 1), :] = jnp.zeros((1, 256), F32)
    xp2[pl.ds(S + 1, 1), :] = jnp.zeros((1, 256), F32)
    xp3[pl.ds(0, 3), :] = jnp.zeros((3, 256), F32)
    xp3[pl.ds(S + 3, 3), :] = jnp.zeros((3, 256), F32)

    b1 = b1_ref[...]
    b2 = b2_ref[...]
    b3 = b3_ref[...]
    bd = bd_ref[...]
    wd = wd_ref[...]

    for r in range(NT):
        r0 = r * TL
        acc = jnp.dot(xp1[pl.ds(r0, TL), :], w1_ref[0],
                      preferred_element_type=F32)
        acc += jnp.dot(xp1[pl.ds(r0 + 1, TL), :], w1_ref[1],
                       preferred_element_type=F32)
        xp2[pl.ds(1 + r0, TL), :] = jnp.maximum(acc + b1, 0.0)
    for r in range(NT):
        r0 = r * TL
        acc = jnp.dot(xp2[pl.ds(r0, TL), :], w2_ref[0],
                      preferred_element_type=F32)
        for w in range(1, 3):
            acc += jnp.dot(xp2[pl.ds(r0 + w, TL), :], w2_ref[w],
                           preferred_element_type=F32)
        xp3[pl.ds(3 + r0, TL), :] = jnp.maximum(acc + b2, 0.0)
    for r in range(NT):
        r0 = r * TL
        acc = jnp.dot(xp3[pl.ds(r0, TL), :], w3_ref[0],
                      preferred_element_type=F32)
        for w in range(1, 4):
            acc += jnp.dot(xp3[pl.ds(r0 + 2 * w, TL), :], w3_ref[w],
                           preferred_element_type=F32)
        h = jnp.maximum(acc + b3, 0.0)
        lg = jnp.dot(h, wd, preferred_element_type=F32)
        out_ref[0, pl.ds(r0, TL), :] = jnp.maximum(lg + bd, 0.0)


def _net(x, w1, b1, w2, b2, w3, b3, wd, bd):
    return pl.pallas_call(
        _net_body,
        grid=(B,),
        in_specs=[
            pl.BlockSpec((1, S, EMB), lambda b: (b, 0, 0)),
            pl.BlockSpec((2, EMB, 256), lambda b: (0, 0, 0)),
            pl.BlockSpec((1, 256), lambda b: (0, 0)),
            pl.BlockSpec((3, 256, 256), lambda b: (0, 0, 0)),
            pl.BlockSpec((1, 256), lambda b: (0, 0)),
            pl.BlockSpec((4, 256, EMB), lambda b: (0, 0, 0)),
            pl.BlockSpec((1, EMB), lambda b: (0, 0)),
            pl.BlockSpec((EMB, C), lambda b: (0, 0)),
            pl.BlockSpec((1, C), lambda b: (0, 0)),
        ],
        out_specs=pl.BlockSpec((1, S, C), lambda b: (b, 0, 0)),
        out_shape=jax.ShapeDtypeStruct((B, S, C), F32),
        scratch_shapes=[
            pltpu.VMEM((S + 1, EMB), F32),
            pltpu.VMEM((S + 2, 256), F32),
            pltpu.VMEM((S + 6, 256), F32),
        ],
        compiler_params=pltpu.CompilerParams(
            dimension_semantics=("arbitrary",)),
    )(x, w1, b1, w2, b2, w3, b3, wd, bd)


def _crf_body(lg_ref, lab_ref, labn_ref, tr_ref, loss_ref, f1_ref, elg_ref,
              a_ref):
    BC = B * C
    trm = tr_ref[...]

    CH = 128
    iota_c = lax.broadcasted_iota(jnp.int32, (CH, C), 1)

    def chunk_step(c, carry):
        emis, pairs, msum, tp, fp, fn = carry
        t0 = c * CH
        es = []
        for b in range(B):
            lgc = lg_ref[b, pl.ds(t0, CH), :]
            l0 = lab_ref[b, pl.ds(t0, CH), :]
            l1 = labn_ref[b, pl.ds(t0, CH), :]
            o0 = (l0 == iota_c)
            o1f = jnp.where(l1 == iota_c, 1.0, 0.0)
            o0f = jnp.where(o0, 1.0, 0.0)
            emis += jnp.sum(jnp.where(o0, lgc, 0.0))
            rowv = jnp.dot(o0f, trm, preferred_element_type=F32)
            pairs += jnp.sum(rowv * o1f)
            mx = jnp.max(lgc, axis=1, keepdims=True)
            pred = jnp.min(jnp.where(lgc == mx, iota_c, jnp.int32(10 ** 9)),
                           axis=1, keepdims=True)
            pf = jnp.where(pred == iota_c, 1.0, 0.0)
            tp += jnp.sum(pf * o0f, axis=0, keepdims=True)
            fp += jnp.sum(pf * (1.0 - o0f), axis=0, keepdims=True)
            fn += jnp.sum((1.0 - pf) * o0f, axis=0, keepdims=True)
            es.append(jnp.exp(lgc - mx))
            msum += jnp.sum(mx)
        elg_ref[pl.ds(t0, CH), :] = jnp.concatenate(es, axis=1)
        return emis, pairs, msum, tp, fp, fn

    zrow = jnp.zeros((1, C), F32)
    emis, pairs, msum, tp, fp, fn = lax.fori_loop(
        0, S // CH, chunk_step,
        (jnp.float32(0.0), jnp.float32(0.0), jnp.float32(0.0),
         zrow, zrow, zrow))

    E = jnp.exp(trm)
    E_rows = jnp.concatenate([E] * B, axis=1)
    E44 = jnp.concatenate([E_rows] * B, axis=0)
    segr = lax.broadcasted_iota(jnp.int32, (BC, BC), 0) // C
    segc = lax.broadcasted_iota(jnp.int32, (BC, BC), 1) // C
    segmask = segr == segc
    E84 = jnp.where(segmask, E44, 0.0)
    eye84 = jnp.where(
        lax.broadcasted_iota(jnp.int32, (BC, BC), 0)
        == lax.broadcasted_iota(jnp.int32, (BC, BC), 1), 1.0, 0.0)
    eye4 = jnp.where(
        lax.broadcasted_iota(jnp.int32, (B, B), 0)
        == lax.broadcasted_iota(jnp.int32, (B, B), 1), 1.0, 0.0)
    ones_seg = jnp.where(
        lax.broadcasted_iota(jnp.int32, (BC, B), 0) // C
        == lax.broadcasted_iota(jnp.int32, (BC, B), 1), 1.0, 0.0)
    spread_seg = jnp.where(
        lax.broadcasted_iota(jnp.int32, (B, BC), 0)
        == lax.broadcasted_iota(jnp.int32, (B, BC), 1) // C, 1.0, 0.0)

    NCK = S // CH
    mlog = jnp.zeros((1, B), F32)

    a_ref[0] = eye84
    for c in range(1, NCK):
        a_ref[c] = E84 * elg_ref[pl.ds(c * CH, 1), :]

    def chain_steps(lo, hi):
        def step(k, _):
            for c in range(NCK):
                e_row = elg_ref[pl.ds(c * CH + k, 1), :]
                a_ref[c] = jnp.dot(a_ref[c], E84,
                                   preferred_element_type=F32) * e_row
            return 0
        lax.fori_loop(lo, hi, step, 0)

    def renorm_chain(c, mlog):
        A = a_ref[c]
        R = jnp.dot(A, ones_seg, preferred_element_type=F32)
        D = jnp.dot(spread_seg, R, preferred_element_type=F32)
        Dd = D * eye4
        s_row = jnp.sum(Dd, axis=0, keepdims=True)
        div = jnp.dot(jnp.dot(ones_seg, Dd, preferred_element_type=F32),
                      spread_seg, preferred_element_type=F32)
        a_ref[c] = A / jnp.where(segmask, div, 1.0)
        return mlog + jnp.log(s_row)

    for g in range(8):
        lo = 1 + 16 * g
        chain_steps(lo, min(lo + 16, CH))
        for c in range(NCK):
            mlog = renorm_chain(c, mlog)

    p = elg_ref[pl.ds(0, 1), :]
    for c in range(NCK):
        p = jnp.dot(p, a_ref[c], preferred_element_type=F32)
        if c % 4 == 3:
            s = jnp.dot(p, ones_seg, preferred_element_type=F32)
            p = p / jnp.dot(s, spread_seg, preferred_element_type=F32)
            mlog = mlog + jnp.log(s)
    tot = jnp.dot(p, ones_seg, preferred_element_type=F32)
    logz_sum = jnp.sum(mlog + jnp.log(tot)) + msum

    loss_ref[0, 0] = (logz_sum - emis - pairs) / B
    p = tp / (tp + fp + 1e-07)
    r = tp / (tp + fn + 1e-07)
    f1 = 2.0 * p * r / (p + r + 1e-07)
    f1_ref[0, 0] = jnp.sum(f1) / C


def _crf(lg, lab3, labn3, trans):
    return pl.pallas_call(
        _crf_body,
        in_specs=[
            pl.BlockSpec((B, S, C), lambda: (0, 0, 0)),
            pl.BlockSpec((B, S, 1), lambda: (0, 0, 0)),
            pl.BlockSpec((B, S, 1), lambda: (0, 0, 0)),
            pl.BlockSpec((C, C), lambda: (0, 0)),
        ],
        out_specs=[
            pl.BlockSpec(memory_space=pltpu.SMEM),
            pl.BlockSpec(memory_space=pltpu.SMEM),
        ],
        out_shape=[
            jax.ShapeDtypeStruct((1, 1), F32),
            jax.ShapeDtypeStruct((1, 1), F32),
        ],
        scratch_shapes=[
            pltpu.VMEM((S, B * C), F32),
            pltpu.VMEM((S // 128, B * C, B * C), F32),
        ],
    )(lg, lab3, labn3, trans)


def kernel(emb_table, conv1_k, conv1_b, conv2_k, conv2_b, idcnn_k, idcnn_b,
           dense_W, dense_b, trans, token_id, label):
    ids = token_id.reshape(B * S).astype(jnp.int32)
    emb = _sc_gather(emb_table, ids)
    x = emb.reshape(B, S, EMB)
    lg = _net(x, conv1_k, conv1_b.reshape(1, -1), conv2_k,
              conv2_b.reshape(1, -1), idcnn_k, idcnn_b.reshape(1, -1),
              dense_W, dense_b.reshape(1, -1))
    lab3 = label.astype(jnp.int32).reshape(B, S, 1)
    labn3 = jnp.concatenate(
        [label.astype(jnp.int32)[:, 1:],
         jnp.full((B, 1), C, jnp.int32)], axis=1).reshape(B, S, 1)
    loss2, f12 = _crf(lg, lab3, labn3, trans)
    return loss2[0, 0], f12[0, 0]
